# Initial kernel scaffold; baseline (speedup 1.0000x reference)
import jax
import jax.numpy as jnp
from jax.experimental import pallas as pl


def kernel(x, edge, W1, b1, W2, b2):
    raise NotImplementedError("write your pallas kernel here")


# speedup vs baseline: 21.2531x; 21.2531x over previous
import functools

import jax
import jax.numpy as jnp
from jax import lax
from jax.experimental import pallas as pl
from jax.experimental.pallas import tpu as pltpu
from jax.experimental.pallas import tpu_sc as plsc

N = 10000
NP = 10240
E = 320000
K = 80
NC, NS = 2, 16
NW = NC * NS
CPW = E // K // NW
RPT = NP // NS
DR = NP // 128


def _sc_mesh():
    return plsc.VectorSubcoreMesh(core_axis_name="c", subcore_axis_name="s")


def _deg_call(dst3, zeros):
    @functools.partial(
        pl.kernel,
        out_type=jax.ShapeDtypeStruct((NW * NP,), jnp.float32),
        mesh=_sc_mesh(),
        compiler_params=pltpu.CompilerParams(needs_layout_passes=False),
        scratch_types=[
            pltpu.VMEM((CPW, K), jnp.int32),
            pltpu.VMEM((NP,), jnp.float32),
        ],
    )
    def deg(dst_hbm, zeros_hbm, out_hbm, idx_v, deg_v):
        c = lax.axis_index("c")
        s = lax.axis_index("s")
        wid = s * NC + c
        def zero(i, carry):
            deg_v[pl.ds(i * 16, 16)] = jnp.zeros((16,), jnp.float32)
            return carry
        lax.fori_loop(0, NP // 16, zero, 0)
        pltpu.sync_copy(dst_hbm.at[wid], idx_v)
        ones = jnp.full((16,), 1.0, jnp.float32)

        def body(j, carry):
            def inner(q, carry2):
                v = idx_v[j, pl.ds(q * 16, 16)]
                plsc.addupdate_scatter(deg_v, [v], ones)
                return carry2
            return lax.fori_loop(0, K // 16, inner, carry)

        lax.fori_loop(0, CPW, body, 0)
        pltpu.sync_copy(deg_v, out_hbm.at[pl.ds(wid * NP, NP)])

    return deg(dst3, zeros)


def _agg_call(u, src3, dst3, zeros):
    @functools.partial(
        pl.kernel,
        out_type=jax.ShapeDtypeStruct((NC, NS, RPT, 128), jnp.float32),
        mesh=_sc_mesh(),
        scratch_types=[
            pltpu.VMEM((CPW, K), jnp.int32),
            pltpu.VMEM((CPW, K), jnp.int32),
            pltpu.VMEM((K, 128), jnp.float32),
            pltpu.VMEM_SHARED((NP, 128), jnp.float32),
            pltpu.SemaphoreType.DMA,
        ],
    )
    def agg(u_hbm, src_hbm, dst_hbm, zeros_hbm, out_hbm,
            src_v, dst_v, rows_v, acc_sp, sem):
        c = lax.axis_index("c")
        s = lax.axis_index("s")
        wid = s * NC + c
        pltpu.sync_copy(zeros_hbm, acc_sp.at[pl.ds(s * RPT, RPT)])
        pltpu.sync_copy(src_hbm.at[wid], src_v)
        pltpu.sync_copy(dst_hbm.at[wid], dst_v)
        plsc.subcore_barrier()

        def body(j, carry):
            pltpu.async_copy(u_hbm.at[src_v.at[j]], rows_v, sem).wait()
            pltpu.sync_copy(rows_v, acc_sp.at[dst_v.at[j]], add=True)
            return carry

        lax.fori_loop(0, CPW, body, 0)
        plsc.subcore_barrier()
        pltpu.sync_copy(acc_sp.at[pl.ds(s * RPT, RPT)], out_hbm.at[c, s])

    return agg(u, src3, dst3, zeros)


def _tc_prep(degp, x):
    BN = 2000

    def body(degp_ref, x_ref, dis_ref, u1_ref):
        deg = jnp.sum(degp_ref[...], axis=1, keepdims=True) + 1.0
        dis = lax.rsqrt(deg)
        dis_ref[...] = dis
        u1_ref[...] = x_ref[...] * dis

    return pl.pallas_call(
        body,
        grid=(N // BN,),
        in_specs=[
            pl.BlockSpec((BN, NW), lambda i: (i, 0)),
            pl.BlockSpec((BN, 128), lambda i: (i, 0)),
        ],
        out_specs=[
            pl.BlockSpec((BN, 1), lambda i: (i, 0)),
            pl.BlockSpec((BN, 128), lambda i: (i, 0)),
        ],
        out_shape=[
            jax.ShapeDtypeStruct((N, 1), jnp.float32),
            jax.ShapeDtypeStruct((N, 128), jnp.float32),
        ],
    )(degp, x)


def _tc_mid(p, u1, dis, W1, b1r, W2p):
    BN = 1000

    def body(p_ref, u1_ref, dis_ref, W1_ref, b1_ref, W2_ref, u2_ref):
        z1 = (p_ref[0] + p_ref[1] + u1_ref[...]) * dis_ref[...]
        h1 = jnp.dot(z1, W1_ref[...], preferred_element_type=jnp.float32)
        h1 = jnp.maximum(h1 + b1_ref[...], 0.0)
        g = jnp.dot(h1, W2_ref[...], preferred_element_type=jnp.float32)
        u2_ref[...] = g * dis_ref[...]

    return pl.pallas_call(
        body,
        grid=(N // BN,),
        in_specs=[
            pl.BlockSpec((NC, BN, 128), lambda i: (0, i, 0)),
            pl.BlockSpec((BN, 128), lambda i: (i, 0)),
            pl.BlockSpec((BN, 1), lambda i: (i, 0)),
            pl.BlockSpec((128, 256), lambda i: (0, 0)),
            pl.BlockSpec((1, 256), lambda i: (0, 0)),
            pl.BlockSpec((256, 128), lambda i: (0, 0)),
        ],
        out_specs=pl.BlockSpec((BN, 128), lambda i: (i, 0)),
        out_shape=jax.ShapeDtypeStruct((N, 128), jnp.float32),
    )(p, u1, dis, W1, b1r, W2p)


def _tc_final(q, u2, dis, b2r):
    BN = 1000

    def body(q_ref, u2_ref, dis_ref, b2_ref, out_ref):
        z = (q_ref[0] + q_ref[1] + u2_ref[...]) * dis_ref[...]
        logits = z[:, :40] + b2_ref[...]
        m = jnp.max(logits, axis=1, keepdims=True)
        ex = jnp.exp(logits - m)
        lse = jnp.log(jnp.sum(ex, axis=1, keepdims=True)) + m
        out_ref[...] = logits - lse

    return pl.pallas_call(
        body,
        grid=(N // BN,),
        in_specs=[
            pl.BlockSpec((NC, BN, 128), lambda i: (0, i, 0)),
            pl.BlockSpec((BN, 128), lambda i: (i, 0)),
            pl.BlockSpec((BN, 1), lambda i: (i, 0)),
            pl.BlockSpec((1, 40), lambda i: (0, 0)),
        ],
        out_specs=pl.BlockSpec((BN, 40), lambda i: (i, 0)),
        out_shape=jax.ShapeDtypeStruct((N, 40), jnp.float32),
    )(q, u2, dis, b2r)


def kernel(x, edge, W1, b1, W2, b2):
    f32 = jnp.float32
    src3 = edge[0].reshape(NW, CPW, K)
    dst3 = edge[1].reshape(NW, CPW, K)
    zeros = jnp.zeros((RPT, 128), f32)
    degp4 = _deg_call(dst3, zeros)
    degp = degp4.reshape(NW, NP)[:, :N].T.reshape(N, NW)
    dis, u1 = _tc_prep(degp, x)
    p4 = _agg_call(u1, src3, dst3, zeros)
    p = p4.reshape(NC, NP, 128)[:, :N]
    u2 = _tc_mid(p, u1, dis, W1, b1.reshape(1, -1),
                 jnp.pad(W2, ((0, 0), (0, 88))))
    q4 = _agg_call(u2, src3, dst3, zeros)
    q = q4.reshape(NC, NP, 128)[:, :N]
    return _tc_final(q, u2, dis, b2.reshape(1, -1))


# speedup vs baseline: 32.0132x; 1.5063x over previous
import functools

import jax
import jax.numpy as jnp
from jax import lax
from jax.experimental import pallas as pl
from jax.experimental.pallas import tpu as pltpu
from jax.experimental.pallas import tpu_sc as plsc

N = 10000
NP = 10240
E = 320000
K = 80
NC, NS = 2, 16
NW = NC * NS
CPW = E // K // NW
RPT = NP // NS
DR = NP // 128


def _sc_mesh():
    return plsc.VectorSubcoreMesh(core_axis_name="c", subcore_axis_name="s")


def _deg_call(dst3, zeros):
    @functools.partial(
        pl.kernel,
        out_type=jax.ShapeDtypeStruct((NW * NP,), jnp.float32),
        mesh=_sc_mesh(),
        compiler_params=pltpu.CompilerParams(needs_layout_passes=False),
        scratch_types=[
            pltpu.VMEM((CPW, K), jnp.int32),
            pltpu.VMEM((NP,), jnp.float32),
        ],
    )
    def deg(dst_hbm, zeros_hbm, out_hbm, idx_v, deg_v):
        c = lax.axis_index("c")
        s = lax.axis_index("s")
        wid = s * NC + c
        def zero(i, carry):
            deg_v[pl.ds(i * 16, 16)] = jnp.zeros((16,), jnp.float32)
            return carry
        lax.fori_loop(0, NP // 16, zero, 0)
        pltpu.sync_copy(dst_hbm.at[wid], idx_v)
        ones = jnp.full((16,), 1.0, jnp.float32)

        def body(j, carry):
            def inner(q, carry2):
                v = idx_v[j, pl.ds(q * 16, 16)]
                plsc.addupdate_scatter(deg_v, [v], ones)
                return carry2
            return lax.fori_loop(0, K // 16, inner, carry)

        lax.fori_loop(0, CPW, body, 0)
        pltpu.sync_copy(deg_v, out_hbm.at[pl.ds(wid * NP, NP)])

    return deg(dst3, zeros)


def _agg_call(u, src3, dst3, zeros):
    @functools.partial(
        pl.kernel,
        out_type=jax.ShapeDtypeStruct((NC, NS, RPT, 128), jnp.float32),
        mesh=_sc_mesh(),
        scratch_types=[
            pltpu.VMEM((CPW * K,), jnp.int32),
            pltpu.VMEM((CPW, K), jnp.int32),
            pltpu.VMEM((K, 128), jnp.float32),
            pltpu.VMEM((K, 128), jnp.float32),
            pltpu.VMEM_SHARED((NP, 128), jnp.float32),
            pltpu.SemaphoreType.DMA,
            pltpu.SemaphoreType.DMA,
        ],
    )
    def agg(u_hbm, src_hbm, dst_hbm, zeros_hbm, out_hbm,
            src_v, dst_v, rows_a, rows_b, acc_sp, sem_a, sem_b):
        c = lax.axis_index("c")
        s = lax.axis_index("s")
        wid = s * NC + c
        pltpu.sync_copy(zeros_hbm, acc_sp.at[pl.ds(s * RPT, RPT)])
        pltpu.sync_copy(src_hbm.at[pl.ds(wid * CPW * K, CPW * K)], src_v)
        pltpu.sync_copy(dst_hbm.at[wid], dst_v)
        plsc.subcore_barrier()

        def gather(j, buf, sem):
            return pltpu.async_copy(
                u_hbm.at[src_v.at[pl.ds(j * K, K)]], buf, sem)

        def wait_gather(buf, sem):
            pltpu.make_async_copy(
                u_hbm.at[src_v.at[pl.ds(0, K)]], buf, sem).wait()

        gather(0, rows_a, sem_a)

        def body(i, carry):
            j = 2 * i
            gather(j + 1, rows_b, sem_b)
            wait_gather(rows_a, sem_a)
            pltpu.sync_copy(rows_a, acc_sp.at[dst_v.at[j]], add=True)
            gather(j + 2, rows_a, sem_a)
            wait_gather(rows_b, sem_b)
            pltpu.sync_copy(rows_b, acc_sp.at[dst_v.at[j + 1]], add=True)
            return carry

        lax.fori_loop(0, CPW // 2, body, 0)
        wait_gather(rows_a, sem_a)
        pltpu.sync_copy(rows_a, acc_sp.at[dst_v.at[CPW - 1]], add=True)
        plsc.subcore_barrier()
        pltpu.sync_copy(acc_sp.at[pl.ds(s * RPT, RPT)], out_hbm.at[c, s])

    return agg(u, src3, dst3, zeros)


def _tc_prep(degp, x):
    BN = 2000

    def body(degp_ref, x_ref, dis_ref, u1_ref):
        deg = jnp.sum(degp_ref[...], axis=1, keepdims=True) + 1.0
        dis = lax.rsqrt(deg)
        dis_ref[...] = dis
        u1_ref[...] = x_ref[...] * dis

    return pl.pallas_call(
        body,
        grid=(N // BN,),
        in_specs=[
            pl.BlockSpec((BN, NW), lambda i: (i, 0)),
            pl.BlockSpec((BN, 128), lambda i: (i, 0)),
        ],
        out_specs=[
            pl.BlockSpec((BN, 1), lambda i: (i, 0)),
            pl.BlockSpec((BN, 128), lambda i: (i, 0)),
        ],
        out_shape=[
            jax.ShapeDtypeStruct((N, 1), jnp.float32),
            jax.ShapeDtypeStruct((N, 128), jnp.float32),
        ],
    )(degp, x)


def _tc_mid(p, u1, dis, W1, b1r, W2p):
    BN = 1000

    def body(p_ref, u1_ref, dis_ref, W1_ref, b1_ref, W2_ref, u2_ref):
        z1 = (p_ref[0] + p_ref[1] + u1_ref[...]) * dis_ref[...]
        h1 = jnp.dot(z1, W1_ref[...], preferred_element_type=jnp.float32)
        h1 = jnp.maximum(h1 + b1_ref[...], 0.0)
        g = jnp.dot(h1, W2_ref[...], preferred_element_type=jnp.float32)
        u2_ref[...] = g * dis_ref[...]

    return pl.pallas_call(
        body,
        grid=(N // BN,),
        in_specs=[
            pl.BlockSpec((NC, BN, 128), lambda i: (0, i, 0)),
            pl.BlockSpec((BN, 128), lambda i: (i, 0)),
            pl.BlockSpec((BN, 1), lambda i: (i, 0)),
            pl.BlockSpec((128, 256), lambda i: (0, 0)),
            pl.BlockSpec((1, 256), lambda i: (0, 0)),
            pl.BlockSpec((256, 128), lambda i: (0, 0)),
        ],
        out_specs=pl.BlockSpec((BN, 128), lambda i: (i, 0)),
        out_shape=jax.ShapeDtypeStruct((N, 128), jnp.float32),
    )(p, u1, dis, W1, b1r, W2p)


def _tc_final(q, u2, dis, b2r):
    BN = 1000

    def body(q_ref, u2_ref, dis_ref, b2_ref, out_ref):
        z = (q_ref[0] + q_ref[1] + u2_ref[...]) * dis_ref[...]
        logits = z[:, :40] + b2_ref[...]
        m = jnp.max(logits, axis=1, keepdims=True)
        ex = jnp.exp(logits - m)
        lse = jnp.log(jnp.sum(ex, axis=1, keepdims=True)) + m
        out_ref[...] = logits - lse

    return pl.pallas_call(
        body,
        grid=(N // BN,),
        in_specs=[
            pl.BlockSpec((NC, BN, 128), lambda i: (0, i, 0)),
            pl.BlockSpec((BN, 128), lambda i: (i, 0)),
            pl.BlockSpec((BN, 1), lambda i: (i, 0)),
            pl.BlockSpec((1, 40), lambda i: (0, 0)),
        ],
        out_specs=pl.BlockSpec((BN, 40), lambda i: (i, 0)),
        out_shape=jax.ShapeDtypeStruct((N, 40), jnp.float32),
    )(q, u2, dis, b2r)


def kernel(x, edge, W1, b1, W2, b2):
    f32 = jnp.float32
    src1 = edge[0]
    dst3 = edge[1].reshape(NW, CPW, K)
    zeros = jnp.zeros((RPT, 128), f32)
    degp4 = _deg_call(dst3, zeros)
    degp = degp4.reshape(NW, NP)[:, :N].T.reshape(N, NW)
    dis, u1 = _tc_prep(degp, x)
    p4 = _agg_call(u1, src1, dst3, zeros)
    p = p4.reshape(NC, NP, 128)[:, :N]
    u2 = _tc_mid(p, u1, dis, W1, b1.reshape(1, -1),
                 jnp.pad(W2, ((0, 0), (0, 88))))
    q4 = _agg_call(u2, src1, dst3, zeros)
    q = q4.reshape(NC, NP, 128)[:, :N]
    return _tc_final(q, u2, dis, b2.reshape(1, -1))


# speedup vs baseline: 34.9476x; 1.0917x over previous
import functools

import jax
import jax.numpy as jnp
from jax import lax
from jax.experimental import pallas as pl
from jax.experimental.pallas import tpu as pltpu
from jax.experimental.pallas import tpu_sc as plsc

N = 10000
NP = 10240
E = 320000
K = 80
NC, NS = 2, 16
NW = NC * NS
CPW = E // K // NW
RPT = NP // NS
DR = NP // 128


def _sc_mesh():
    return plsc.VectorSubcoreMesh(core_axis_name="c", subcore_axis_name="s")


def _deg_call(dst3, zeros):
    @functools.partial(
        pl.kernel,
        out_type=jax.ShapeDtypeStruct((NW * NP,), jnp.float32),
        mesh=_sc_mesh(),
        compiler_params=pltpu.CompilerParams(needs_layout_passes=False),
        scratch_types=[
            pltpu.VMEM((CPW, K), jnp.int32),
            pltpu.VMEM((NP,), jnp.float32),
        ],
    )
    def deg(dst_hbm, zeros_hbm, out_hbm, idx_v, deg_v):
        c = lax.axis_index("c")
        s = lax.axis_index("s")
        wid = s * NC + c
        def zero(i, carry):
            deg_v[pl.ds(i * 16, 16)] = jnp.zeros((16,), jnp.float32)
            return carry
        lax.fori_loop(0, NP // 16, zero, 0)
        pltpu.sync_copy(dst_hbm.at[wid], idx_v)
        ones = jnp.full((16,), 1.0, jnp.float32)

        def body(j, carry):
            def inner(q, carry2):
                v = idx_v[j, pl.ds(q * 16, 16)]
                plsc.addupdate_scatter(deg_v, [v], ones)
                return carry2
            return lax.fori_loop(0, K // 16, inner, carry)

        lax.fori_loop(0, CPW, body, 0)
        pltpu.sync_copy(deg_v, out_hbm.at[pl.ds(wid * NP, NP)])

    return deg(dst3, zeros)


def _agg_call(u, src3, dst3, zeros, D=128, tc_tiling=True):
    @functools.partial(
        pl.kernel,
        out_type=jax.ShapeDtypeStruct((NC, NS, RPT, D), jnp.float32),
        mesh=_sc_mesh(),
        compiler_params=pltpu.CompilerParams(use_tc_tiling_on_sc=tc_tiling),
        scratch_types=[
            pltpu.VMEM((CPW * K,), jnp.int32),
            pltpu.VMEM((CPW, K), jnp.int32),
            pltpu.VMEM((K, D), jnp.float32),
            pltpu.VMEM((K, D), jnp.float32),
            pltpu.VMEM_SHARED((NP, D), jnp.float32),
            pltpu.SemaphoreType.DMA,
            pltpu.SemaphoreType.DMA,
        ],
    )
    def agg(u_hbm, src_hbm, dst_hbm, zeros_hbm, out_hbm,
            src_v, dst_v, rows_a, rows_b, acc_sp, sem_a, sem_b):
        c = lax.axis_index("c")
        s = lax.axis_index("s")
        wid = s * NC + c
        pltpu.sync_copy(zeros_hbm, acc_sp.at[pl.ds(s * RPT, RPT)])
        pltpu.sync_copy(src_hbm.at[pl.ds(wid * CPW * K, CPW * K)], src_v)
        pltpu.sync_copy(dst_hbm.at[wid], dst_v)
        plsc.subcore_barrier()

        def gather(j, buf, sem):
            return pltpu.async_copy(
                u_hbm.at[src_v.at[pl.ds(j * K, K)]], buf, sem)

        def wait_gather(buf, sem):
            pltpu.make_async_copy(
                u_hbm.at[src_v.at[pl.ds(0, K)]], buf, sem).wait()

        gather(0, rows_a, sem_a)

        def body(i, carry):
            j = 2 * i
            gather(j + 1, rows_b, sem_b)
            wait_gather(rows_a, sem_a)
            pltpu.sync_copy(rows_a, acc_sp.at[dst_v.at[j]], add=True)
            gather(j + 2, rows_a, sem_a)
            wait_gather(rows_b, sem_b)
            pltpu.sync_copy(rows_b, acc_sp.at[dst_v.at[j + 1]], add=True)
            return carry

        lax.fori_loop(0, CPW // 2, body, 0)
        wait_gather(rows_a, sem_a)
        pltpu.sync_copy(rows_a, acc_sp.at[dst_v.at[CPW - 1]], add=True)
        plsc.subcore_barrier()
        pltpu.sync_copy(acc_sp.at[pl.ds(s * RPT, RPT)], out_hbm.at[c, s])

    return agg(u, src3, dst3, zeros)


def _tc_prep(degp, x):
    BN = 2000

    def body(degp_ref, x_ref, dis_ref, u1_ref):
        deg = jnp.sum(degp_ref[...], axis=1, keepdims=True) + 1.0
        dis = lax.rsqrt(deg)
        dis_ref[...] = dis
        u1_ref[...] = x_ref[...] * dis

    return pl.pallas_call(
        body,
        grid=(N // BN,),
        in_specs=[
            pl.BlockSpec((BN, NW), lambda i: (i, 0)),
            pl.BlockSpec((BN, 128), lambda i: (i, 0)),
        ],
        out_specs=[
            pl.BlockSpec((BN, 1), lambda i: (i, 0)),
            pl.BlockSpec((BN, 128), lambda i: (i, 0)),
        ],
        out_shape=[
            jax.ShapeDtypeStruct((N, 1), jnp.float32),
            jax.ShapeDtypeStruct((N, 128), jnp.float32),
        ],
    )(degp, x)


def _tc_mid(p, u1, dis, W1, b1r, W2p):
    BN = 1000

    def body(p_ref, u1_ref, dis_ref, W1_ref, b1_ref, W2_ref, u2_ref):
        z1 = (p_ref[0] + p_ref[1] + u1_ref[...]) * dis_ref[...]
        h1 = jnp.dot(z1, W1_ref[...], preferred_element_type=jnp.float32)
        h1 = jnp.maximum(h1 + b1_ref[...], 0.0)
        g = jnp.dot(h1, W2_ref[...], preferred_element_type=jnp.float32)
        u2_ref[...] = g * dis_ref[...]

    return pl.pallas_call(
        body,
        grid=(N // BN,),
        in_specs=[
            pl.BlockSpec((NC, BN, 128), lambda i: (0, i, 0)),
            pl.BlockSpec((BN, 128), lambda i: (i, 0)),
            pl.BlockSpec((BN, 1), lambda i: (i, 0)),
            pl.BlockSpec((128, 256), lambda i: (0, 0)),
            pl.BlockSpec((1, 256), lambda i: (0, 0)),
            pl.BlockSpec((256, 48), lambda i: (0, 0)),
        ],
        out_specs=pl.BlockSpec((BN, 48), lambda i: (i, 0)),
        out_shape=jax.ShapeDtypeStruct((N, 48), jnp.float32),
    )(p, u1, dis, W1, b1r, W2p)


def _tc_final(q, u2, dis, b2r):
    BN = 1000

    def body(q_ref, u2_ref, dis_ref, b2_ref, out_ref):
        z = (q_ref[0] + q_ref[1] + u2_ref[...]) * dis_ref[...]
        logits = z[:, :40] + b2_ref[...]
        m = jnp.max(logits, axis=1, keepdims=True)
        ex = jnp.exp(logits - m)
        lse = jnp.log(jnp.sum(ex, axis=1, keepdims=True)) + m
        out_ref[...] = logits - lse

    return pl.pallas_call(
        body,
        grid=(N // BN,),
        in_specs=[
            pl.BlockSpec((NC, BN, 48), lambda i: (0, i, 0)),
            pl.BlockSpec((BN, 48), lambda i: (i, 0)),
            pl.BlockSpec((BN, 1), lambda i: (i, 0)),
            pl.BlockSpec((1, 40), lambda i: (0, 0)),
        ],
        out_specs=pl.BlockSpec((BN, 40), lambda i: (i, 0)),
        out_shape=jax.ShapeDtypeStruct((N, 40), jnp.float32),
    )(q, u2, dis, b2r)


def kernel(x, edge, W1, b1, W2, b2):
    f32 = jnp.float32
    src1 = edge[0]
    dst3 = edge[1].reshape(NW, CPW, K)
    zeros = jnp.zeros((RPT, 128), f32)
    degp4 = _deg_call(dst3, zeros)
    degp = degp4.reshape(NW, NP)[:, :N].T.reshape(N, NW)
    dis, u1 = _tc_prep(degp, x)
    p4 = _agg_call(u1, src1, dst3, zeros)
    p = p4.reshape(NC, NP, 128)[:, :N]
    u2 = _tc_mid(p, u1, dis, W1, b1.reshape(1, -1),
                 jnp.pad(W2, ((0, 0), (0, 8))))
    q4 = _agg_call(u2, src1, dst3, jnp.zeros((RPT, 48), f32),
                   D=48, tc_tiling=False)
    q = q4.reshape(NC, NP, 48)[:, :N]
    return _tc_final(q, u2, dis, b2.reshape(1, -1))


# speedup vs baseline: 36.4859x; 1.0440x over previous
import functools

import jax
import jax.numpy as jnp
from jax import lax
from jax.experimental import pallas as pl
from jax.experimental.pallas import tpu as pltpu
from jax.experimental.pallas import tpu_sc as plsc

N = 10000
E = 320000
K = 80
NC, NS = 2, 16
NW = NC * NS
CPW = E // K // NW
RF = 632
RL = N - (NS - 1) * RF


def _sc_mesh():
    return plsc.VectorSubcoreMesh(core_axis_name="c", subcore_axis_name="s")


def _deg_call(dst3, zeros):
    @functools.partial(
        pl.kernel,
        out_type=jax.ShapeDtypeStruct((NW * N,), jnp.float32),
        mesh=_sc_mesh(),
        compiler_params=pltpu.CompilerParams(needs_layout_passes=False),
        scratch_types=[
            pltpu.VMEM((CPW, K), jnp.int32),
            pltpu.VMEM((N,), jnp.float32),
        ],
    )
    def deg(dst_hbm, zeros_hbm, out_hbm, idx_v, deg_v):
        c = lax.axis_index("c")
        s = lax.axis_index("s")
        wid = s * NC + c
        def zero(i, carry):
            deg_v[pl.ds(i * 16, 16)] = jnp.zeros((16,), jnp.float32)
            return carry
        lax.fori_loop(0, N // 16, zero, 0)
        pltpu.sync_copy(dst_hbm.at[wid], idx_v)
        ones = jnp.full((16,), 1.0, jnp.float32)

        def body(j, carry):
            def inner(q, carry2):
                v = idx_v[j, pl.ds(q * 16, 16)]
                plsc.addupdate_scatter(deg_v, [v], ones)
                return carry2
            return lax.fori_loop(0, K // 16, inner, carry)

        lax.fori_loop(0, CPW, body, 0)
        pltpu.sync_copy(deg_v, out_hbm.at[pl.ds(wid * N, N)])

    return deg(dst3, zeros)


def _agg_call(u, src3, dst3, zeros, D=128, tc_tiling=True):
    @functools.partial(
        pl.kernel,
        out_type=jax.ShapeDtypeStruct((NC, N, D), jnp.float32),
        mesh=_sc_mesh(),
        compiler_params=pltpu.CompilerParams(use_tc_tiling_on_sc=tc_tiling),
        scratch_types=[
            pltpu.VMEM((CPW * K,), jnp.int32),
            pltpu.VMEM((CPW, K), jnp.int32),
            pltpu.VMEM((K, D), jnp.float32),
            pltpu.VMEM((K, D), jnp.float32),
            pltpu.VMEM_SHARED((N, D), jnp.float32),
            pltpu.SemaphoreType.DMA,
            pltpu.SemaphoreType.DMA,
        ],
    )
    def agg(u_hbm, src_hbm, dst_hbm, zeros_hbm, out_hbm,
            src_v, dst_v, rows_a, rows_b, acc_sp, sem_a, sem_b):
        c = lax.axis_index("c")
        s = lax.axis_index("s")
        wid = s * NC + c

        @pl.when(s < NS - 1)
        def _():
            pltpu.sync_copy(zeros_hbm.at[pl.ds(0, RF)],
                            acc_sp.at[pl.ds(s * RF, RF)])

        @pl.when(s == NS - 1)
        def _():
            pltpu.sync_copy(zeros_hbm.at[pl.ds(0, RL)],
                            acc_sp.at[pl.ds((NS - 1) * RF, RL)])

        pltpu.sync_copy(src_hbm.at[pl.ds(wid * CPW * K, CPW * K)], src_v)
        pltpu.sync_copy(dst_hbm.at[wid], dst_v)
        plsc.subcore_barrier()

        def gather(j, buf, sem):
            return pltpu.async_copy(
                u_hbm.at[src_v.at[pl.ds(j * K, K)]], buf, sem)

        def wait_gather(buf, sem):
            pltpu.make_async_copy(
                u_hbm.at[src_v.at[pl.ds(0, K)]], buf, sem).wait()

        gather(0, rows_a, sem_a)

        def body(i, carry):
            j = 2 * i
            gather(j + 1, rows_b, sem_b)
            wait_gather(rows_a, sem_a)
            pltpu.sync_copy(rows_a, acc_sp.at[dst_v.at[j]], add=True)
            gather(j + 2, rows_a, sem_a)
            wait_gather(rows_b, sem_b)
            pltpu.sync_copy(rows_b, acc_sp.at[dst_v.at[j + 1]], add=True)
            return carry

        lax.fori_loop(0, CPW // 2, body, 0)
        wait_gather(rows_a, sem_a)
        pltpu.sync_copy(rows_a, acc_sp.at[dst_v.at[CPW - 1]], add=True)
        plsc.subcore_barrier()

        @pl.when(s < NS - 1)
        def _():
            pltpu.sync_copy(acc_sp.at[pl.ds(s * RF, RF)],
                            out_hbm.at[c, pl.ds(s * RF, RF)])

        @pl.when(s == NS - 1)
        def _():
            pltpu.sync_copy(acc_sp.at[pl.ds((NS - 1) * RF, RL)],
                            out_hbm.at[c, pl.ds((NS - 1) * RF, RL)])

    return agg(u, src3, dst3, zeros)


def _tc_prep(degp, x):
    BN = 2000

    def body(degp_ref, x_ref, dis_ref, u1_ref):
        deg = jnp.sum(degp_ref[...], axis=1, keepdims=True) + 1.0
        dis = lax.rsqrt(deg)
        dis_ref[...] = dis
        u1_ref[...] = x_ref[...] * dis

    return pl.pallas_call(
        body,
        grid=(N // BN,),
        in_specs=[
            pl.BlockSpec((BN, NW), lambda i: (i, 0)),
            pl.BlockSpec((BN, 128), lambda i: (i, 0)),
        ],
        out_specs=[
            pl.BlockSpec((BN, 1), lambda i: (i, 0)),
            pl.BlockSpec((BN, 128), lambda i: (i, 0)),
        ],
        out_shape=[
            jax.ShapeDtypeStruct((N, 1), jnp.float32),
            jax.ShapeDtypeStruct((N, 128), jnp.float32),
        ],
    )(degp, x)


def _tc_mid(p, u1, dis, W1, b1r, W2p):
    BN = 1000

    def body(p_ref, u1_ref, dis_ref, W1_ref, b1_ref, W2_ref, u2_ref):
        z1 = (p_ref[0] + p_ref[1] + u1_ref[...]) * dis_ref[...]
        h1 = jnp.dot(z1, W1_ref[...], preferred_element_type=jnp.float32)
        h1 = jnp.maximum(h1 + b1_ref[...], 0.0)
        g = jnp.dot(h1, W2_ref[...], preferred_element_type=jnp.float32)
        u2_ref[...] = g * dis_ref[...]

    return pl.pallas_call(
        body,
        grid=(N // BN,),
        in_specs=[
            pl.BlockSpec((NC, BN, 128), lambda i: (0, i, 0)),
            pl.BlockSpec((BN, 128), lambda i: (i, 0)),
            pl.BlockSpec((BN, 1), lambda i: (i, 0)),
            pl.BlockSpec((128, 256), lambda i: (0, 0)),
            pl.BlockSpec((1, 256), lambda i: (0, 0)),
            pl.BlockSpec((256, 48), lambda i: (0, 0)),
        ],
        out_specs=pl.BlockSpec((BN, 48), lambda i: (i, 0)),
        out_shape=jax.ShapeDtypeStruct((N, 48), jnp.float32),
    )(p, u1, dis, W1, b1r, W2p)


def _tc_final(q, u2, dis, b2r):
    BN = 1000

    def body(q_ref, u2_ref, dis_ref, b2_ref, out_ref):
        z = (q_ref[0] + q_ref[1] + u2_ref[...]) * dis_ref[...]
        logits = z[:, :40] + b2_ref[...]
        m = jnp.max(logits, axis=1, keepdims=True)
        ex = jnp.exp(logits - m)
        lse = jnp.log(jnp.sum(ex, axis=1, keepdims=True)) + m
        out_ref[...] = logits - lse

    return pl.pallas_call(
        body,
        grid=(N // BN,),
        in_specs=[
            pl.BlockSpec((NC, BN, 48), lambda i: (0, i, 0)),
            pl.BlockSpec((BN, 48), lambda i: (i, 0)),
            pl.BlockSpec((BN, 1), lambda i: (i, 0)),
            pl.BlockSpec((1, 40), lambda i: (0, 0)),
        ],
        out_specs=pl.BlockSpec((BN, 40), lambda i: (i, 0)),
        out_shape=jax.ShapeDtypeStruct((N, 40), jnp.float32),
    )(q, u2, dis, b2r)


def kernel(x, edge, W1, b1, W2, b2):
    f32 = jnp.float32
    src1 = edge[0]
    dst3 = edge[1].reshape(NW, CPW, K)
    zeros = jnp.zeros((RF, 128), f32)
    degf = _deg_call(dst3, zeros)
    degp = degf.reshape(NW, N).T.reshape(N, NW)
    dis, u1 = _tc_prep(degp, x)
    p = _agg_call(u1, src1, dst3, zeros)
    u2 = _tc_mid(p, u1, dis, W1, b1.reshape(1, -1),
                 jnp.pad(W2, ((0, 0), (0, 8))))
    q = _agg_call(u2, src1, dst3, jnp.zeros((RF, 48), f32),
                  D=48, tc_tiling=False)
    return _tc_final(q, u2, dis, b2.reshape(1, -1))
